# baseline (speedup 1.0000x reference)
import functools

import jax
import jax.numpy as jnp
from jax.experimental import pallas as pl

_HID = 1024
_LAT = 512
_H = 16
_HD = 64
_W = 128
_P = 16
_FORGET = 0.1
_NEG = float(jnp.finfo(jnp.float32).min)
_BQ = 128
_G = 1
_SEARCH_BITS = 22

_keep_cache = {}


def _keep_mask(h, s):
    key = (h, s)
    if key not in _keep_cache:
        with jax.ensure_compile_time_eval():
            fkey = jax.random.key(42)
            r = jax.random.uniform(fkey, (1, h, s, s)) >= _FORGET
            col = jnp.arange(s)[None, :]
            row = jnp.arange(s)[:, None]
            causal = col <= row
            force = ((col < _P) & causal) | (col == row)
            v = (causal.astype(jnp.int8)
                 + 2 * (r[0] & causal).astype(jnp.int8)
                 + 4 * force.astype(jnp.int8))
            _keep_cache[key] = v
    return _keep_cache[key]


def _proj_body(x_ref, wq_ref, wc_ref, wk_ref, wv_ref, q_ref, k_ref, v_ref):
    x = x_ref[...]
    qb = jnp.dot(x, wq_ref[...], preferred_element_type=jnp.float32)
    c = jnp.dot(x, wc_ref[...], preferred_element_type=jnp.float32)
    kb = jnp.dot(c, wk_ref[...], preferred_element_type=jnp.float32)
    vb = jnp.dot(c, wv_ref[...], preferred_element_type=jnp.float32)
    for h in range(_H):
        sl = slice(h * _HD, (h + 1) * _HD)
        q_ref[h] = qb[:, sl]
        k_ref[h] = kb[:, sl]
        v_ref[h] = vb[:, sl]


def _attn_body(q_ref, k_ref, v_ref, rnd_ref, o_ref, *, bq, w, scale):
    q = q_ref[0] * scale
    k = k_ref[0]
    sc = jax.lax.dot_general(q, k, (((1,), (1,)), ((), ())),
                             preferred_element_type=jnp.float32)
    vb = rnd_ref[0].astype(jnp.int32)
    bits = jax.lax.bitcast_convert_type(sc, jnp.uint32)
    u = jnp.where(sc >= 0, bits | jnp.uint32(0x80000000), ~bits)
    u = jnp.where(vb != 0, u, jnp.uint32(0))

    def body(i, prefix):
        cand = prefix | (jnp.uint32(1) << (31 - i).astype(jnp.uint32))
        cnt = jnp.sum((u >= cand).astype(jnp.int32), axis=1, keepdims=True)
        return jnp.where(cnt >= _W, cand, prefix)

    kth = jax.lax.fori_loop(0, _SEARCH_BITS, body,
                            jnp.zeros((bq, 1), jnp.uint32))
    keep = ((u >= kth) & (vb >= 3)) | (vb >= 5)
    m = jnp.max(jnp.where(keep, sc, _NEG), axis=1, keepdims=True)
    p = jnp.where(keep, jnp.exp(sc - m), 0.0)
    denom = jnp.sum(p, axis=1, keepdims=True)
    o_ref[0] = jnp.dot(p / denom, v_ref[0],
                       preferred_element_type=jnp.float32)


def _outproj_body(a_ref, wo_ref, o_ref):
    xcat = jnp.concatenate([a_ref[h] for h in range(_H)], axis=1)
    o_ref[...] = jnp.dot(xcat, wo_ref[...],
                         preferred_element_type=jnp.float32)


def kernel(x, Wq, Wc, Wk, Wv, Wo):
    b, s, d = x.shape
    x2 = x.reshape(s, d)
    q, k, v = pl.pallas_call(
        _proj_body,
        grid=(s // _BQ,),
        in_specs=[pl.BlockSpec((_BQ, d), lambda i: (i, 0)),
                  pl.BlockSpec((d, d), lambda i: (0, 0)),
                  pl.BlockSpec((d, _LAT), lambda i: (0, 0)),
                  pl.BlockSpec((_LAT, d), lambda i: (0, 0)),
                  pl.BlockSpec((_LAT, d), lambda i: (0, 0))],
        out_specs=[pl.BlockSpec((_H, _BQ, _HD), lambda i: (0, i, 0))] * 3,
        out_shape=[jax.ShapeDtypeStruct((_H, s, _HD), jnp.float32)] * 3,
    )(x2, Wq, Wc, Wk, Wv)
    rnd = _keep_mask(_H, s)
    nqb = s // _BQ
    outs = []
    for g in range(nqb // _G):
        row0 = g * _G * _BQ
        w = row0 + _G * _BQ
        outs.append(pl.pallas_call(
            functools.partial(_attn_body, bq=_BQ, w=w, scale=_HD ** -0.5),
            grid=(_H, _G),
            in_specs=[
                pl.BlockSpec((1, _BQ, _HD),
                             lambda h, i, g=g: (h, g * _G + i, 0)),
                pl.BlockSpec((1, w, _HD), lambda h, i: (h, 0, 0)),
                pl.BlockSpec((1, w, _HD), lambda h, i: (h, 0, 0)),
                pl.BlockSpec((1, _BQ, w),
                             lambda h, i, g=g: (h, g * _G + i, 0)),
            ],
            out_specs=pl.BlockSpec((1, _BQ, _HD), lambda h, i: (h, i, 0)),
            out_shape=jax.ShapeDtypeStruct((_H, _G * _BQ, _HD), jnp.float32),
        )(q, k, v, rnd))
    attn_out = jnp.concatenate(outs, axis=1)
    out = pl.pallas_call(
        _outproj_body,
        grid=(s // _BQ,),
        in_specs=[pl.BlockSpec((_H, _BQ, _HD), lambda i: (0, i, 0)),
                  pl.BlockSpec((d, d), lambda i: (0, 0))],
        out_specs=pl.BlockSpec((_BQ, d), lambda i: (i, 0)),
        out_shape=jax.ShapeDtypeStruct((s, d), jnp.float32),
    )(attn_out, Wo)
    return out.reshape(b, s, d)


# speedup vs baseline: 1.4942x; 1.4942x over previous
import functools

import jax
import jax.numpy as jnp
from jax.experimental import pallas as pl

_HID = 1024
_LAT = 512
_H = 16
_HD = 64
_W = 128
_P = 16
_FORGET = 0.1
_NEG = float(jnp.finfo(jnp.float32).min)
_BQ = 512
_G = 1
_SEARCH_BITS = 22

_keep_cache = {}


def _keep_mask(h, s):
    key = (h, s)
    if key not in _keep_cache:
        with jax.ensure_compile_time_eval():
            fkey = jax.random.key(42)
            r = jax.random.uniform(fkey, (1, h, s, s)) >= _FORGET
            col = jnp.arange(s)[None, :]
            row = jnp.arange(s)[:, None]
            causal = col <= row
            force = ((col < _P) & causal) | (col == row)
            v = (causal.astype(jnp.int8)
                 + 2 * (r[0] & causal).astype(jnp.int8)
                 + 4 * force.astype(jnp.int8))
            _keep_cache[key] = v
    return _keep_cache[key]


def _proj_body(x_ref, wq_ref, wc_ref, wk_ref, wv_ref, q_ref, k_ref, v_ref):
    x = x_ref[...]
    qb = jnp.dot(x, wq_ref[...], preferred_element_type=jnp.float32)
    c = jnp.dot(x, wc_ref[...], preferred_element_type=jnp.float32)
    kb = jnp.dot(c, wk_ref[...], preferred_element_type=jnp.float32)
    vb = jnp.dot(c, wv_ref[...], preferred_element_type=jnp.float32)
    for h in range(_H):
        sl = slice(h * _HD, (h + 1) * _HD)
        q_ref[h] = qb[:, sl]
        k_ref[h] = kb[:, sl]
        v_ref[h] = vb[:, sl]


def _attn_body(q_ref, k_ref, v_ref, rnd_ref, o_ref, *, bq, w, scale):
    q = q_ref[0] * scale
    k = k_ref[0]
    sc = jax.lax.dot_general(q, k, (((1,), (1,)), ((), ())),
                             preferred_element_type=jnp.float32)
    vb = rnd_ref[0].astype(jnp.int32)
    bits = jax.lax.bitcast_convert_type(sc, jnp.uint32)
    u = jnp.where(sc >= 0, bits | jnp.uint32(0x80000000), ~bits)
    u = jnp.where(vb != 0, u, jnp.uint32(0))

    def body(i, prefix):
        cand = prefix | (jnp.uint32(1) << (31 - i).astype(jnp.uint32))
        cnt = jnp.sum((u >= cand).astype(jnp.int32), axis=1, keepdims=True)
        return jnp.where(cnt >= _W, cand, prefix)

    kth = jax.lax.fori_loop(0, _SEARCH_BITS, body,
                            jnp.zeros((bq, 1), jnp.uint32))
    keep = ((u >= kth) & (vb >= 3)) | (vb >= 5)
    m = jnp.max(jnp.where(keep, sc, _NEG), axis=1, keepdims=True)
    p = jnp.where(keep, jnp.exp(sc - m), 0.0)
    denom = jnp.sum(p, axis=1, keepdims=True)
    o_ref[0] = jnp.dot(p / denom, v_ref[0],
                       preferred_element_type=jnp.float32)


def _outproj_body(a_ref, wo_ref, o_ref):
    xcat = jnp.concatenate([a_ref[h] for h in range(_H)], axis=1)
    o_ref[...] = jnp.dot(xcat, wo_ref[...],
                         preferred_element_type=jnp.float32)


def kernel(x, Wq, Wc, Wk, Wv, Wo):
    b, s, d = x.shape
    x2 = x.reshape(s, d)
    q, k, v = pl.pallas_call(
        _proj_body,
        grid=(s // _BQ,),
        in_specs=[pl.BlockSpec((_BQ, d), lambda i: (i, 0)),
                  pl.BlockSpec((d, d), lambda i: (0, 0)),
                  pl.BlockSpec((d, _LAT), lambda i: (0, 0)),
                  pl.BlockSpec((_LAT, d), lambda i: (0, 0)),
                  pl.BlockSpec((_LAT, d), lambda i: (0, 0))],
        out_specs=[pl.BlockSpec((_H, _BQ, _HD), lambda i: (0, i, 0))] * 3,
        out_shape=[jax.ShapeDtypeStruct((_H, s, _HD), jnp.float32)] * 3,
    )(x2, Wq, Wc, Wk, Wv)
    rnd = _keep_mask(_H, s)
    nqb = s // _BQ
    outs = []
    for g in range(nqb // _G):
        row0 = g * _G * _BQ
        w = row0 + _G * _BQ
        outs.append(pl.pallas_call(
            functools.partial(_attn_body, bq=_BQ, w=w, scale=_HD ** -0.5),
            grid=(_H, _G),
            in_specs=[
                pl.BlockSpec((1, _BQ, _HD),
                             lambda h, i, g=g: (h, g * _G + i, 0)),
                pl.BlockSpec((1, w, _HD), lambda h, i: (h, 0, 0)),
                pl.BlockSpec((1, w, _HD), lambda h, i: (h, 0, 0)),
                pl.BlockSpec((1, _BQ, w),
                             lambda h, i, g=g: (h, g * _G + i, 0)),
            ],
            out_specs=pl.BlockSpec((1, _BQ, _HD), lambda h, i: (h, i, 0)),
            out_shape=jax.ShapeDtypeStruct((_H, _G * _BQ, _HD), jnp.float32),
        )(q, k, v, rnd))
    attn_out = jnp.concatenate(outs, axis=1)
    out = pl.pallas_call(
        _outproj_body,
        grid=(s // _BQ,),
        in_specs=[pl.BlockSpec((_H, _BQ, _HD), lambda i: (0, i, 0)),
                  pl.BlockSpec((d, d), lambda i: (0, 0))],
        out_specs=pl.BlockSpec((_BQ, d), lambda i: (i, 0)),
        out_shape=jax.ShapeDtypeStruct((s, d), jnp.float32),
    )(attn_out, Wo)
    return out.reshape(b, s, d)
